# Initial kernel scaffold; baseline (speedup 1.0000x reference)
import jax
import jax.numpy as jnp
from jax.experimental import pallas as pl


def kernel(x, edge_index, W_l, b_l, W_r, b_r, conv_w, conv_b, attn, alpha_act):
    raise NotImplementedError("write your pallas kernel here")


# speedup vs baseline: 78.0999x; 78.0999x over previous
import functools

import jax
import jax.numpy as jnp
from jax import lax
from jax.experimental import pallas as pl
from jax.experimental.pallas import tpu as pltpu
from jax.experimental.pallas import tpu_sc as plsc

N = 10000
E = 320000
D = 128
H = 4
C = D // H
TW = 144
CHUNK = 80
NTILES = 32
EDGES_PER_TILE = E // NTILES
CHUNKS_PER_TILE = EDGES_PER_TILE // CHUNK
NPAD = 10240
ROWS_PER_TILE = NPAD // 16
ZROWS = 128


def _prep_body(x_ref, wl_ref, bl_ref, wr_ref, br_ref, mj_ref,
               table_ref, r_ref):
    x = x_ref[...]
    dn = (((1,), (1,)), ((), ()))
    l = lax.dot_general(x, wl_ref[...], dn,
                        preferred_element_type=jnp.float32) + bl_ref[...]
    r = lax.dot_general(x, wr_ref[...], dn,
                        preferred_element_type=jnp.float32) + br_ref[...]
    dn2 = (((1,), (0,)), ((), ()))
    aj = lax.dot_general(l, mj_ref[...], dn2,
                         preferred_element_type=jnp.float32)
    w = jnp.exp(jnp.clip(aj, -60.0, 60.0))
    wfull = jnp.concatenate(
        [jnp.broadcast_to(w[:, h:h + 1], (x.shape[0], C)) for h in range(H)],
        axis=1)
    lw = l * wfull
    pad = jnp.zeros((x.shape[0], TW - D - H), dtype=jnp.float32)
    table_ref[...] = jnp.concatenate([lw, w[:, :H], pad], axis=1)
    r_ref[...] = r


def _sc_edge_body(src_hbm, dst_hbm, table_hbm, out_hbm,
                  src_v, dst_v, rows_v, zrow_v, accum_sh):
    c = lax.axis_index("c")
    s = lax.axis_index("s")
    tile = s * 2 + c

    def zb(i, _):
        for j in range(TW // 16):
            zrow_v[i, pl.ds(j * 16, 16)] = jnp.zeros((16,), jnp.float32)
        return 0
    lax.fori_loop(0, ZROWS, zb, 0)
    row0 = s * ROWS_PER_TILE
    def zc(k, _):
        pltpu.sync_copy(zrow_v, accum_sh.at[pl.ds(row0 + k * ZROWS, ZROWS)])
        return 0
    lax.fori_loop(0, ROWS_PER_TILE // ZROWS, zc, 0)
    plsc.subcore_barrier()

    ebase = tile * EDGES_PER_TILE

    def body(j, _):
        off = ebase + j * CHUNK
        pltpu.sync_copy(src_hbm.at[pl.ds(off, CHUNK)], src_v)
        pltpu.sync_copy(dst_hbm.at[pl.ds(off, CHUNK)], dst_v)
        pltpu.sync_copy(table_hbm.at[src_v], rows_v)
        pltpu.sync_copy(rows_v, accum_sh.at[dst_v], add=True)
        return 0
    lax.fori_loop(0, CHUNKS_PER_TILE, body, 0)

    plsc.subcore_barrier()
    pltpu.sync_copy(accum_sh.at[pl.ds(row0, ROWS_PER_TILE)],
                    out_hbm.at[c, pl.ds(row0, ROWS_PER_TILE)])


def _final_body(p_ref, r_ref, cw_ref, cb_ref, out_ref):
    s2 = p_ref[0] + p_ref[1]
    r = r_ref[...]
    nb = r.shape[0]
    denom = s2[:, D:D + H]
    dfull = jnp.concatenate(
        [jnp.broadcast_to(denom[:, h:h + 1], (nb, C)) for h in range(H)],
        axis=1)
    agg = s2[:, :D] / jnp.maximum(dfull, 1e-30)
    dn = (((1,), (1,)), ((), ()))
    logits = lax.dot_general(r, cw_ref[...], dn,
                             preferred_element_type=jnp.float32) + cb_ref[...]
    m = jnp.max(logits, axis=1, keepdims=True)
    e = jnp.exp(logits - m)
    beta = e / jnp.sum(e, axis=1, keepdims=True)
    out = beta[:, 0:1] * agg + beta[:, 1:2] * r
    out_ref[...] = jnp.maximum(out, 0.0)


def kernel(x, edge_index, W_l, b_l, W_r, b_r, conv_w, conv_b, attn, alpha_act):
    s = alpha_act[0]
    attnj = attn[0, :, C:].reshape(-1)
    head_of = jnp.arange(D, dtype=jnp.int32) // C
    mj = (head_of[:, None] == jnp.arange(8, dtype=jnp.int32)[None, :])
    mj = mj.astype(jnp.float32) * (attnj * s)[:, None]

    table, r = pl.pallas_call(
        _prep_body,
        out_shape=(jax.ShapeDtypeStruct((N, TW), jnp.float32),
                   jax.ShapeDtypeStruct((N, D), jnp.float32)),
    )(x, W_l, b_l.reshape(1, D), W_r, b_r.reshape(1, D), mj)

    mesh = plsc.VectorSubcoreMesh(core_axis_name="c", subcore_axis_name="s")
    partials = pl.kernel(
        _sc_edge_body,
        mesh=mesh,
        compiler_params=pltpu.CompilerParams(use_tc_tiling_on_sc=False),
        out_type=jax.ShapeDtypeStruct((2, NPAD, TW), jnp.float32),
        scratch_types=[
            pltpu.VMEM((CHUNK,), jnp.int32),
            pltpu.VMEM((CHUNK,), jnp.int32),
            pltpu.VMEM((CHUNK, TW), jnp.float32),
            pltpu.VMEM((ZROWS, TW), jnp.float32),
            pltpu.VMEM_SHARED((NPAD, TW), jnp.float32),
        ],
    )(edge_index[0], edge_index[1], table)

    NB = 2000
    out = pl.pallas_call(
        _final_body,
        grid=(N // NB,),
        in_specs=[
            pl.BlockSpec((2, NB, TW), lambda i: (0, i, 0)),
            pl.BlockSpec((NB, D), lambda i: (i, 0)),
            pl.BlockSpec((2, D), lambda i: (0, 0)),
            pl.BlockSpec((1, 2), lambda i: (0, 0)),
        ],
        out_specs=pl.BlockSpec((NB, D), lambda i: (i, 0)),
        out_shape=jax.ShapeDtypeStruct((N, D), jnp.float32),
    )(partials, r, conv_w, conv_b.reshape(1, 2))
    return out


# speedup vs baseline: 126.6793x; 1.6220x over previous
import functools

import jax
import jax.numpy as jnp
from jax import lax
from jax.experimental import pallas as pl
from jax.experimental.pallas import tpu as pltpu
from jax.experimental.pallas import tpu_sc as plsc

N = 10000
E = 320000
D = 128
H = 4
C = D // H
TW = 144
CHUNK = 80
NTILES = 32
EDGES_PER_TILE = E // NTILES
CHUNKS_PER_TILE = EDGES_PER_TILE // CHUNK
NPAD = 10240
ROWS_PER_TILE = NPAD // 16


def _prep_body(x_ref, wl_ref, bl_ref, wr_ref, br_ref, mj_ref,
               table_ref, r_ref):
    x = x_ref[...]
    dn = (((1,), (1,)), ((), ()))
    l = lax.dot_general(x, wl_ref[...], dn,
                        preferred_element_type=jnp.float32) + bl_ref[...]
    r = lax.dot_general(x, wr_ref[...], dn,
                        preferred_element_type=jnp.float32) + br_ref[...]
    dn2 = (((1,), (0,)), ((), ()))
    aj = lax.dot_general(l, mj_ref[...], dn2,
                         preferred_element_type=jnp.float32)
    w = jnp.exp(jnp.clip(aj, -60.0, 60.0))
    wfull = jnp.concatenate(
        [jnp.broadcast_to(w[:, h:h + 1], (x.shape[0], C)) for h in range(H)],
        axis=1)
    lw = l * wfull
    pad = jnp.zeros((x.shape[0], TW - D - H), dtype=jnp.float32)
    table_ref[...] = jnp.concatenate([lw, w[:, :H], pad], axis=1)
    r_ref[...] = r


def _sc_edge_body(src_hbm, dst_hbm, table_hbm, out_hbm,
                  src_v, dst_v, rows_v, rows2_v, accum_sh, gsem):
    c = lax.axis_index("c")
    s = lax.axis_index("s")
    tile = s * 2 + c

    def zb(i, _):
        for j in range(TW // 16):
            rows_v[i, pl.ds(j * 16, 16)] = jnp.zeros((16,), jnp.float32)
        return 0
    lax.fori_loop(0, CHUNK, zb, 0)
    row0 = s * ROWS_PER_TILE
    def zc(k, _):
        pltpu.sync_copy(rows_v, accum_sh.at[pl.ds(row0 + k * CHUNK, CHUNK)])
        return 0
    lax.fori_loop(0, ROWS_PER_TILE // CHUNK, zc, 0)
    plsc.subcore_barrier()

    rbase = tile * CHUNKS_PER_TILE
    for pbase, psize in ((0, 63), (63, 31), (94, 31)):
        pltpu.sync_copy(src_hbm.at[pl.ds(rbase + pbase, psize)],
                        src_v.at[pl.ds(0, psize)])
        pltpu.sync_copy(dst_hbm.at[pl.ds(rbase + pbase, psize)],
                        dst_v.at[pl.ds(0, psize)])
        pltpu.async_copy(table_hbm.at[src_v.at[0]], rows_v, gsem).wait()

        def body(i, _):
            j = 2 * i
            cp = pltpu.async_copy(table_hbm.at[src_v.at[j + 1]], rows2_v,
                                  gsem)
            pltpu.sync_copy(rows_v, accum_sh.at[dst_v.at[j]], add=True)
            cp.wait()
            cp2 = pltpu.async_copy(table_hbm.at[src_v.at[j + 2]], rows_v,
                                   gsem)
            pltpu.sync_copy(rows2_v, accum_sh.at[dst_v.at[j + 1]], add=True)
            cp2.wait()
            return 0
        lax.fori_loop(0, (psize - 1) // 2, body, 0)
        pltpu.sync_copy(rows_v, accum_sh.at[dst_v.at[psize - 1]], add=True)

    plsc.subcore_barrier()
    pltpu.sync_copy(accum_sh.at[pl.ds(row0, ROWS_PER_TILE)],
                    out_hbm.at[c, pl.ds(row0, ROWS_PER_TILE)])


def _final_body(p_ref, r_ref, cw_ref, cb_ref, out_ref):
    s2 = p_ref[0] + p_ref[1]
    r = r_ref[...]
    nb = r.shape[0]
    denom = s2[:, D:D + H]
    dfull = jnp.concatenate(
        [jnp.broadcast_to(denom[:, h:h + 1], (nb, C)) for h in range(H)],
        axis=1)
    agg = s2[:, :D] / jnp.maximum(dfull, 1e-30)
    dn = (((1,), (1,)), ((), ()))
    logits = lax.dot_general(r, cw_ref[...], dn,
                             preferred_element_type=jnp.float32) + cb_ref[...]
    m = jnp.max(logits, axis=1, keepdims=True)
    e = jnp.exp(logits - m)
    beta = e / jnp.sum(e, axis=1, keepdims=True)
    out = beta[:, 0:1] * agg + beta[:, 1:2] * r
    out_ref[...] = jnp.maximum(out, 0.0)


def kernel(x, edge_index, W_l, b_l, W_r, b_r, conv_w, conv_b, attn, alpha_act):
    s = alpha_act[0]
    attnj = attn[0, :, C:].reshape(-1)
    head_of = jnp.arange(D, dtype=jnp.int32) // C
    mj = (head_of[:, None] == jnp.arange(8, dtype=jnp.int32)[None, :])
    mj = mj.astype(jnp.float32) * (attnj * s)[:, None]

    table, r = pl.pallas_call(
        _prep_body,
        out_shape=(jax.ShapeDtypeStruct((N, TW), jnp.float32),
                   jax.ShapeDtypeStruct((N, D), jnp.float32)),
    )(x, W_l, b_l.reshape(1, D), W_r, b_r.reshape(1, D), mj)

    mesh = plsc.VectorSubcoreMesh(core_axis_name="c", subcore_axis_name="s")
    partials = pl.kernel(
        _sc_edge_body,
        mesh=mesh,
        compiler_params=pltpu.CompilerParams(use_tc_tiling_on_sc=False),
        out_type=jax.ShapeDtypeStruct((2, NPAD, TW), jnp.float32),
        scratch_types=[
            pltpu.VMEM((63, CHUNK), jnp.int32),
            pltpu.VMEM((63, CHUNK), jnp.int32),
            pltpu.VMEM((CHUNK, TW), jnp.float32),
            pltpu.VMEM((CHUNK, TW), jnp.float32),
            pltpu.VMEM_SHARED((NPAD, TW), jnp.float32),
            pltpu.SemaphoreType.DMA,
        ],
    )(edge_index[0].reshape(NTILES * CHUNKS_PER_TILE, CHUNK),
      edge_index[1].reshape(NTILES * CHUNKS_PER_TILE, CHUNK),
      table)

    NB = 2000
    out = pl.pallas_call(
        _final_body,
        grid=(N // NB,),
        in_specs=[
            pl.BlockSpec((2, NB, TW), lambda i: (0, i, 0)),
            pl.BlockSpec((NB, D), lambda i: (i, 0)),
            pl.BlockSpec((2, D), lambda i: (0, 0)),
            pl.BlockSpec((1, 2), lambda i: (0, 0)),
        ],
        out_specs=pl.BlockSpec((NB, D), lambda i: (i, 0)),
        out_shape=jax.ShapeDtypeStruct((N, D), jnp.float32),
    )(partials, r, conv_w, conv_b.reshape(1, 2))
    return out
